# baseline (speedup 1.0000x reference)
import functools

import jax
import jax.numpy as jnp
from jax import lax
from jax.experimental import pallas as pl
from jax.experimental.pallas import tpu as pltpu
from jax.experimental.pallas import tpu_sc as plsc

N = 10000
E = 320000
D = 128

NC = 2
NS = 16
K = 80
SUP = 5

EPC = E // NC
EPT = EPC // NS
ROWS_T = EPT // K
NSUP = ROWS_T // SUP
FL = 640
ZR = 80
CW = 16


_DEBUG_NO_COUNTS = True
_DEBUG_NO_STREAMS = True
_DEBUG_XLA_SEGSUM = True


def _xla_segsum(table, er, num_sets, with_counts):
    ei = er.reshape(num_sets, 2, -1)
    epc = ei.shape[2] // NC
    sums, cnts = [], []
    for s in range(num_sets):
        per_core = []
        for c in range(NC):
            src = ei[s, 0, c * epc:(c + 1) * epc]
            dst = ei[s, 1, c * epc:(c + 1) * epc]
            per_core.append(jax.ops.segment_sum(table[src], dst, num_segments=N))
        sums.append(jnp.stack(per_core))
        cc = jax.ops.segment_sum(jnp.ones((ei.shape[2],), jnp.float32),
                                 ei[s, 1], num_segments=N)
        cnts.append(jnp.broadcast_to(cc[:, None], (N, CW)))
    if with_counts:
        return jnp.stack(sums), jnp.stack(cnts)
    return (jnp.stack(sums),)


def _segsum_body(num_sets, with_counts, table, edges, sums_out, cnts_out,
                 idx_s, idx_d, idx_d2, rows, ones_b, zcnt, acc, cnt, sem):
    cid = lax.axis_index("c")
    sid = lax.axis_index("s")

    z16 = jnp.zeros((16,), jnp.float32)
    o16 = jnp.ones((16,), jnp.float32)

    def _fill_z(i, _):
        for j in range(D // 16):
            rows[i, pl.ds(16 * j, 16)] = z16
        return ()

    def _fill_zc(i, _):
        zcnt[i, pl.ds(0, 16)] = z16
        return ()
    lax.fori_loop(0, ZR, _fill_zc, ())

    if with_counts:
        def _fill_o(i, _):
            ones_b[i, pl.ds(0, 16)] = o16
            return ()
        lax.fori_loop(0, K, _fill_o, ())

    nch = jnp.where(sid == NS - 1, (N - FL * (NS - 1)) // ZR, FL // ZR)

    def _zero_acc(j, _):
        pltpu.sync_copy(rows, acc.at[pl.ds(FL * sid + ZR * j, ZR), :])
        return ()

    def _zero_cnt(j, _):
        pltpu.sync_copy(zcnt, cnt.at[pl.ds(FL * sid + ZR * j, ZR), :])
        return ()

    lax.fori_loop(0, K, _fill_z, ())
    if with_counts:
        lax.fori_loop(0, nch, _zero_cnt, ())
    lax.fori_loop(0, nch, _zero_acc, ())
    plsc.subcore_barrier()

    def _per_set(s, _):
        cc = jnp.where(s < 2, 0, 1)
        count_here = (cid == cc) if with_counts else None

        def _super(sc, _):
            pltpu.sync_copy(edges.at[s, 0, cid, sid, sc], idx_s)
            pltpu.sync_copy(edges.at[s, 1, cid, sid, sc], idx_d)

            if with_counts and not _DEBUG_NO_COUNTS:
                @pl.when(count_here)
                def _():
                    pltpu.sync_copy(edges.at[s, 1, 1 - cc, sid, sc], idx_d2)

            def _stream(j, _):
                if not _DEBUG_NO_STREAMS:
                    pltpu.async_copy(table.at[idx_s.at[j]], rows, sem).wait()
                    pltpu.sync_copy(rows, acc.at[idx_d.at[j]], add=True)

                if with_counts and not _DEBUG_NO_COUNTS:
                    @pl.when(count_here)
                    def _():
                        pltpu.sync_copy(ones_b, cnt.at[idx_d.at[j]], add=True)
                        pltpu.sync_copy(ones_b, cnt.at[idx_d2.at[j]], add=True)
                return ()
            lax.fori_loop(0, SUP, _stream, ())
            return ()
        lax.fori_loop(0, NSUP, _super, ())
        plsc.subcore_barrier()

        def _flush(j, _):
            off = FL * sid + ZR * j
            pltpu.sync_copy(acc.at[pl.ds(off, ZR), :],
                            sums_out.at[s, cid, pl.ds(off, ZR), :])

            if with_counts:
                @pl.when(count_here)
                def _():
                    pltpu.sync_copy(cnt.at[pl.ds(off, ZR), :],
                                    cnts_out.at[s, pl.ds(off, ZR), :])
            return ()
        lax.fori_loop(0, nch, _flush, ())
        if num_sets > 1:
            lax.fori_loop(0, K, _fill_z, ())
            lax.fori_loop(0, nch, _zero_acc, ())
            if with_counts:
                lax.fori_loop(0, nch, _zero_cnt, ())
            plsc.subcore_barrier()
        return ()

    lax.fori_loop(0, num_sets, _per_set, ())


def _make_segsum(num_sets, with_counts):
    out_type = [jax.ShapeDtypeStruct((num_sets, NC, N, D), jnp.float32)]
    if with_counts:
        out_type.append(jax.ShapeDtypeStruct((num_sets, N, CW), jnp.float32))

    scratch = [
        pltpu.VMEM((SUP, K), jnp.int32),
        pltpu.VMEM((SUP, K), jnp.int32),
        pltpu.VMEM((K, D), jnp.float32),
        pltpu.VMEM((ZR, CW), jnp.float32),
        pltpu.VMEM_SHARED((N, D), jnp.float32),
    ]
    if with_counts:
        scratch += [
            pltpu.VMEM((SUP, K), jnp.int32),
            pltpu.VMEM((K, CW), jnp.float32),
            pltpu.VMEM_SHARED((N, CW), jnp.float32),
        ]
    scratch.append(pltpu.SemaphoreType.DMA)

    def body(table, edges, *rest):
        if with_counts:
            (sums_out, cnts_out, idx_s, idx_d, rows, zcnt, acc,
             idx_d2, ones_b, cnt, sem) = rest
        else:
            (sums_out, idx_s, idx_d, rows, zcnt, acc, sem) = rest
            cnts_out = idx_d2 = ones_b = cnt = None
        _segsum_body(num_sets, with_counts, table, edges, sums_out, cnts_out,
                     idx_s, idx_d, idx_d2, rows, ones_b, zcnt, acc,
                     cnt, sem)

    mesh = plsc.VectorSubcoreMesh(core_axis_name="c", subcore_axis_name="s")
    return pl.kernel(body, out_type=tuple(out_type), mesh=mesh,
                     scratch_types=scratch)


_make_segsum = functools.lru_cache(maxsize=None)(_make_segsum)


def _tc_b_body(x_ref, sums_ref, cnts_ref,
               wl0_ref, wr0_ref, b0_ref, wtl0_ref, wtr0_ref, bt0_ref,
               wtl1_ref, wtr1_ref, bt1_ref, wg0_ref, bg0_ref, wg1_ref, bg1_ref,
               wr1_ref, b1_ref, out0_ref, part_ref):
    x = x_ref[...]
    f32 = jnp.float32

    def mean(s):
        tot = sums_ref[s, 0] + sums_ref[s, 1]
        cnt = cnts_ref[s, :, 0:1]
        return tot / jnp.maximum(cnt, 1.0)

    m0, mt0, mt1 = mean(0), mean(1), mean(2)

    out0 = jnp.maximum(
        jnp.dot(m0, wl0_ref[...], preferred_element_type=f32)
        + jnp.dot(x, wr0_ref[...], preferred_element_type=f32) + b0_ref[...], 0.0)
    t0 = jnp.maximum(
        jnp.dot(mt0, wtl0_ref[...], preferred_element_type=f32)
        + jnp.dot(x, wtr0_ref[...], preferred_element_type=f32) + bt0_ref[...], 0.0)
    t1 = jnp.maximum(
        jnp.dot(mt1, wtl1_ref[...], preferred_element_type=f32)
        + jnp.dot(x, wtr1_ref[...], preferred_element_type=f32) + bt1_ref[...], 0.0)

    w0 = jnp.sum(t0 * wg0_ref[...], axis=1, keepdims=True) + bg0_ref[...]
    w1 = jnp.sum(t1 * wg1_ref[...], axis=1, keepdims=True) + bg1_ref[...]
    a0 = 1.0 / (1.0 + jnp.exp(w1 - w0))
    x_tree = t0 * a0 + t1 * (1.0 - a0)

    out0_ref[...] = out0
    part_ref[...] = (jnp.dot(out0, wr1_ref[...], preferred_element_type=f32)
                     + b1_ref[...] + x_tree)


def _tc_c_body(s1_ref, cnts_ref, part_ref, wl1_ref, out_ref):
    tot = s1_ref[0] + s1_ref[1]
    cnt = cnts_ref[0, :, 0:1]
    m1 = tot / jnp.maximum(cnt, 1.0)
    out_ref[...] = (jnp.dot(m1, wl1_ref[...], preferred_element_type=jnp.float32)
                    + part_ref[...])


def kernel(x, edge_index, W_l0, W_r0, b0, W_l1, W_r1, b1,
           Wt_l0, Wt_r0, bt0, Wt_l1, Wt_r1, bt1, Wg0, bg0, Wg1, bg1):
    f32 = jnp.float32
    er = edge_index.reshape(3, 2, NC, NS, NSUP, SUP, K)

    if _DEBUG_XLA_SEGSUM:
        sums, cnts = _xla_segsum(x, er, 3, True)
    else:
        sums, cnts = _make_segsum(3, True)(x, er)

    BN = 1000
    grid = (N // BN,)

    def full(shape):
        return pl.BlockSpec(shape, lambda i: (0,) * len(shape))

    tc_b = pl.pallas_call(
        _tc_b_body,
        grid=grid,
        in_specs=[
            pl.BlockSpec((BN, D), lambda i: (i, 0)),
            pl.BlockSpec((3, NC, BN, D), lambda i: (0, 0, i, 0)),
            pl.BlockSpec((3, BN, CW), lambda i: (0, i, 0)),
            full((D, D)), full((D, D)), full((1, D)),
            full((D, D)), full((D, D)), full((1, D)),
            full((D, D)), full((D, D)), full((1, D)),
            full((1, D)), full((1, 1)),
            full((1, D)), full((1, 1)),
            full((D, D)), full((1, D)),
        ],
        out_specs=[pl.BlockSpec((BN, D), lambda i: (i, 0)),
                   pl.BlockSpec((BN, D), lambda i: (i, 0))],
        out_shape=[jax.ShapeDtypeStruct((N, D), f32),
                   jax.ShapeDtypeStruct((N, D), f32)],
    )
    out0, part = tc_b(
        x, sums, cnts,
        W_l0, W_r0, b0.reshape(1, D), Wt_l0, Wt_r0, bt0.reshape(1, D),
        Wt_l1, Wt_r1, bt1.reshape(1, D), Wg0.reshape(1, D), bg0.reshape(1, 1),
        Wg1.reshape(1, D), bg1.reshape(1, 1), W_r1, b1.reshape(1, D))

    if _DEBUG_XLA_SEGSUM:
        (s1,) = _xla_segsum(out0, er[0:1], 1, False)
    else:
        (s1,) = _make_segsum(1, False)(out0, er[0:1])

    tc_c = pl.pallas_call(
        _tc_c_body,
        grid=grid,
        in_specs=[
            pl.BlockSpec((NC, BN, D), lambda i: (0, i, 0)),
            pl.BlockSpec((1, BN, CW), lambda i: (0, i, 0)),
            pl.BlockSpec((BN, D), lambda i: (i, 0)),
            full((D, D)),
        ],
        out_specs=pl.BlockSpec((BN, D), lambda i: (i, 0)),
        out_shape=jax.ShapeDtypeStruct((N, D), f32),
    )
    return tc_c(s1[0], cnts[0:1], part, W_l1)


# speedup vs baseline: 3.8766x; 3.8766x over previous
import functools

import jax
import jax.numpy as jnp
from jax import lax
from jax.experimental import pallas as pl
from jax.experimental.pallas import tpu as pltpu
from jax.experimental.pallas import tpu_sc as plsc

N = 10000
E = 320000
D = 128

NC = 2
NS = 16
K = 125

EPT = E // (NC * NS)
ROWS_T = EPT // K
CH = 16
NCHK = ROWS_T // CH
FL = 640
ZR = 80
NCH = FL // ZR
NPAD = NS * FL


def _segsum_body(num_sets, with_counts, table, ones_t, es, ed, sums_out,
                 cnts_out, idx_s, idx_d, rows_a, rows_b, acc, sem):
    sid = lax.axis_index("s")
    cid = lax.axis_index("c")
    blk0 = cid * NS + sid

    z16 = jnp.zeros((16,), jnp.float32)

    def _fill_z():
        def _f(i, _):
            for j in range(D // 16):
                rows_a[i, pl.ds(16 * j, 16)] = z16
            return ()
        pl.loop(0, K)(lambda i: (_f(i, ()), None)[1])

    def _chunk_off(j):
        return pl.multiple_of(FL * sid + ZR * j, 8)

    def _zero_acc(j, _):
        pltpu.async_copy(rows_a.at[pl.ds(0, ZR), :],
                         acc.at[pl.ds(_chunk_off(j), ZR), :], sem).wait()
        return ()

    def _zero():
        _fill_z()
        pl.loop(0, NCH)(lambda j: (_zero_acc(j, ()), None)[1])

    def _flush(dst, s):
        def _fl(j, _):
            off = _chunk_off(j)
            pltpu.sync_copy(acc.at[pl.ds(off, ZR), :],
                            dst.at[s, cid, pl.ds(off, ZR), :])
            return ()
        pl.loop(0, NCH)(lambda j: (_fl(j, ()), None)[1])

    def _pass(tab, s, use_src):
        def _chunk(c, _):
            if use_src:
                pltpu.sync_copy(es.at[s * NC * NS + blk0, pl.ds(c * CH, CH)],
                                idx_s)
            pltpu.sync_copy(ed.at[s * NC * NS + blk0, pl.ds(c * CH, CH)],
                            idx_d)
            gidx = idx_s if use_src else idx_d

            def _stream2(j, _):
                pltpu.async_copy(tab.at[gidx.at[2 * j]], rows_a, sem).wait()
                pltpu.sync_copy(rows_a, acc.at[idx_d.at[2 * j]], add=True)
                pltpu.async_copy(tab.at[gidx.at[2 * j + 1]], rows_b,
                                 sem).wait()
                pltpu.sync_copy(rows_b, acc.at[idx_d.at[2 * j + 1]], add=True)
                return ()
            pl.loop(0, CH // 2)(lambda j: (_stream2(j, ()), None)[1])
            return ()
        pl.loop(0, NCHK)(lambda c: (_chunk(c, ()), None)[1])

    _zero()
    plsc.subcore_barrier()

    for s in range(num_sets):
        _pass(table, s, True)
        plsc.subcore_barrier()
        _flush(sums_out, s)
        _zero()
        plsc.subcore_barrier()

        if with_counts:
            _pass(ones_t, s, False)
            plsc.subcore_barrier()
            _flush(cnts_out, s)
            if s + 1 < num_sets:
                _zero()
                plsc.subcore_barrier()


def _make_segsum(num_sets, with_counts):
    out_type = [jax.ShapeDtypeStruct((num_sets, NC, NPAD, D), jnp.float32)]
    if with_counts:
        out_type.append(
            jax.ShapeDtypeStruct((num_sets, NC, NPAD, D), jnp.float32))

    scratch = [
        pltpu.VMEM((CH, K), jnp.int32),
        pltpu.VMEM((CH, K), jnp.int32),
        pltpu.VMEM((K, D), jnp.float32),
        pltpu.VMEM((K, D), jnp.float32),
        pltpu.VMEM_SHARED((NPAD, D), jnp.float32),
        pltpu.SemaphoreType.DMA,
    ]

    def body(*args):
        if with_counts:
            (table, ones_t, es, ed, sums_out, cnts_out,
             idx_s, idx_d, rows_a, rows_b, acc, sem) = args
        else:
            (table, es, ed, sums_out,
             idx_s, idx_d, rows_a, rows_b, acc, sem) = args
            ones_t = cnts_out = None
        _segsum_body(num_sets, with_counts, table, ones_t, es, ed, sums_out,
                     cnts_out, idx_s, idx_d, rows_a, rows_b, acc, sem)

    mesh = plsc.VectorSubcoreMesh(core_axis_name="c", subcore_axis_name="s")
    return pl.kernel(body, out_type=tuple(out_type), mesh=mesh,
                     scratch_types=scratch)


_make_segsum = functools.lru_cache(maxsize=None)(_make_segsum)


def _tc_b_body(x_ref, sums_ref, cnts_ref,
               wl0_ref, wr0_ref, b0_ref, wtl0_ref, wtr0_ref, bt0_ref,
               wtl1_ref, wtr1_ref, bt1_ref, wg0_ref, bg0_ref, wg1_ref, bg1_ref,
               wr1_ref, b1_ref, out0_ref, part_ref):
    x = x_ref[...]
    f32 = jnp.float32

    def mean(s):
        tot = sums_ref[s, 0] + sums_ref[s, 1]
        cnt = cnts_ref[s, 0, :, 0:1] + cnts_ref[s, 1, :, 0:1]
        return tot / jnp.maximum(cnt, 1.0)

    m0, mt0, mt1 = mean(0), mean(1), mean(2)

    out0 = jnp.maximum(
        jnp.dot(m0, wl0_ref[...], preferred_element_type=f32)
        + jnp.dot(x, wr0_ref[...], preferred_element_type=f32) + b0_ref[...], 0.0)
    t0 = jnp.maximum(
        jnp.dot(mt0, wtl0_ref[...], preferred_element_type=f32)
        + jnp.dot(x, wtr0_ref[...], preferred_element_type=f32) + bt0_ref[...], 0.0)
    t1 = jnp.maximum(
        jnp.dot(mt1, wtl1_ref[...], preferred_element_type=f32)
        + jnp.dot(x, wtr1_ref[...], preferred_element_type=f32) + bt1_ref[...], 0.0)

    w0 = jnp.sum(t0 * wg0_ref[...], axis=1, keepdims=True) + bg0_ref[...]
    w1 = jnp.sum(t1 * wg1_ref[...], axis=1, keepdims=True) + bg1_ref[...]
    a0 = 1.0 / (1.0 + jnp.exp(w1 - w0))
    x_tree = t0 * a0 + t1 * (1.0 - a0)

    out0_ref[...] = out0
    part_ref[...] = (jnp.dot(out0, wr1_ref[...], preferred_element_type=f32)
                     + b1_ref[...] + x_tree)


def _tc_c_body(s1_ref, cnts_ref, part_ref, wl1_ref, out_ref):
    tot = s1_ref[0] + s1_ref[1]
    cnt = cnts_ref[0, 0, :, 0:1] + cnts_ref[0, 1, :, 0:1]
    m1 = tot / jnp.maximum(cnt, 1.0)
    out_ref[...] = (jnp.dot(m1, wl1_ref[...], preferred_element_type=jnp.float32)
                    + part_ref[...])


def kernel(x, edge_index, W_l0, W_r0, b0, W_l1, W_r1, b1,
           Wt_l0, Wt_r0, bt0, Wt_l1, Wt_r1, bt1, Wg0, bg0, Wg1, bg1):
    f32 = jnp.float32
    er = edge_index.reshape(3, 2, NC * NS, ROWS_T, K)
    es = er[:, 0].reshape(3 * NC * NS, ROWS_T, K)
    ed = er[:, 1].reshape(3 * NC * NS, ROWS_T, K)

    ones_t = jnp.ones((N, D), f32)
    sums_p, cnts_p = _make_segsum(3, True)(x, ones_t, es, ed)
    sums = sums_p[:, :, :N, :]
    cnts = cnts_p[:, :, :N, :]

    BN = 1000
    grid = (N // BN,)

    def full(shape):
        return pl.BlockSpec(shape, lambda i: (0,) * len(shape))

    tc_b = pl.pallas_call(
        _tc_b_body,
        grid=grid,
        in_specs=[
            pl.BlockSpec((BN, D), lambda i: (i, 0)),
            pl.BlockSpec((3, NC, BN, D), lambda i: (0, 0, i, 0)),
            pl.BlockSpec((3, NC, BN, D), lambda i: (0, 0, i, 0)),
            full((D, D)), full((D, D)), full((1, D)),
            full((D, D)), full((D, D)), full((1, D)),
            full((D, D)), full((D, D)), full((1, D)),
            full((1, D)), full((1, 1)),
            full((1, D)), full((1, 1)),
            full((D, D)), full((1, D)),
        ],
        out_specs=[pl.BlockSpec((BN, D), lambda i: (i, 0)),
                   pl.BlockSpec((BN, D), lambda i: (i, 0))],
        out_shape=[jax.ShapeDtypeStruct((N, D), f32),
                   jax.ShapeDtypeStruct((N, D), f32)],
    )
    out0, part = tc_b(
        x, sums, cnts,
        W_l0, W_r0, b0.reshape(1, D), Wt_l0, Wt_r0, bt0.reshape(1, D),
        Wt_l1, Wt_r1, bt1.reshape(1, D), Wg0.reshape(1, D), bg0.reshape(1, 1),
        Wg1.reshape(1, D), bg1.reshape(1, 1), W_r1, b1.reshape(1, D))

    s1_p = _make_segsum(1, False)(out0, es[0:NC * NS], ed[0:NC * NS])[0]
    s1 = s1_p[:, :, :N, :]

    tc_c = pl.pallas_call(
        _tc_c_body,
        grid=grid,
        in_specs=[
            pl.BlockSpec((NC, BN, D), lambda i: (0, i, 0)),
            pl.BlockSpec((1, NC, BN, D), lambda i: (0, 0, i, 0)),
            pl.BlockSpec((BN, D), lambda i: (i, 0)),
            full((D, D)),
        ],
        out_specs=pl.BlockSpec((BN, D), lambda i: (i, 0)),
        out_shape=jax.ShapeDtypeStruct((N, D), f32),
    )
    return tc_c(s1[0], cnts[0:1], part, W_l1)
